# baseline (speedup 1.0000x reference)
import dataclasses
import functools

import jax
import jax.numpy as jnp
from jax import lax
from jax.experimental import pallas as pl
from jax.experimental.pallas import tpu as pltpu
from jax.experimental.pallas import tpu_sc as plsc

NUM_CORES = 2
NUM_SUBCORES = 16
LANES = 16


def _make_sc_spmm(n_nodes: int, e_pad: int):
    assert e_pad % (NUM_SUBCORES * 2048) == 0
    groups_per_tile = e_pad // 512 // NUM_SUBCORES
    stages = groups_per_tile // 4
    rows_a = ((n_nodes // NUM_SUBCORES) + 7) // 8 * 8
    rows_b = n_nodes - (NUM_SUBCORES - 1) * rows_a
    assert rows_b > 0 and rows_b % 8 == 0
    acc_rows = ((n_nodes + 1 + 16 * 448 - 1) // (16 * 448)) * (16 * 448)
    z_rows = acc_rows // NUM_SUBCORES // 448

    mesh = plsc.VectorSubcoreMesh(
        core_axis_name="c", subcore_axis_name="s",
        num_cores=NUM_CORES, num_subcores=NUM_SUBCORES)

    cp = pltpu.CompilerParams()
    if "needs_layout_passes" in pltpu.CompilerParams.__dataclass_fields__:
        cp = dataclasses.replace(cp, needs_layout_passes=False)
    if "use_tc_tiling_on_sc" in pltpu.CompilerParams.__dataclass_fields__:
        cp = dataclasses.replace(cp, use_tc_tiling_on_sc=False)

    @functools.partial(
        pl.kernel,
        compiler_params=cp,
        out_type=jax.ShapeDtypeStruct((4, n_nodes, 16), jnp.float32),
        mesh=mesh,
        scratch_types=[
            pltpu.VMEM((4, 512), jnp.int32),
            pltpu.VMEM((4, 512), jnp.int32),
            pltpu.VMEM((4, 512), jnp.float32),
            pltpu.VMEM((512, 16), jnp.float32),
            pltpu.VMEM_SHARED((acc_rows, 16), jnp.float32),
            pltpu.VMEM_SHARED((acc_rows, 16), jnp.float32),
            pltpu.SemaphoreType.DMA,
            pltpu.SemaphoreType.DMA,
        ],
    )
    def sc_spmm(x4_hbm, src_hbm, dst_hbm, vals_hbm, out_hbm,
                src_st, dst_st, vals_st, rows, tbl, acc, gsem, ssem):
        c = lax.axis_index("c")
        s = lax.axis_index("s")
        zv = jnp.zeros((LANES,), jnp.float32)
        lbase = s * rows_a

        dnums = lax.GatherDimensionNumbers(
            offset_dims=(), collapsed_slice_dims=(0,), start_index_map=(0,))
        bcast_idx = [jnp.full((LANES, 1), i, jnp.int32) for i in range(16)]

        @pl.loop(0, 512)
        def _(i):
            rows[i, :] = zv

        for p in range(2):
            q = c * 2 + p

            @pl.when(s < NUM_SUBCORES - 1)
            def _():
                pltpu.sync_copy(x4_hbm.at[q, pl.ds(lbase, rows_a)],
                                tbl.at[pl.ds(lbase, rows_a)])

            @pl.when(s == NUM_SUBCORES - 1)
            def _():
                pltpu.sync_copy(x4_hbm.at[q, pl.ds(lbase, rows_b)],
                                tbl.at[pl.ds(lbase, rows_b)])

            zbase = s * (z_rows * 448)

            @pl.loop(0, z_rows)
            def _(k):
                pltpu.sync_copy(rows.at[pl.ds(0, 448)],
                                acc.at[pl.ds(zbase + k * 448, 448)])

            plsc.subcore_barrier()

            row0 = s * groups_per_tile

            @pl.loop(0, stages)
            def _(st):
                r0 = row0 + st * 4
                pltpu.sync_copy(src_hbm.at[pl.ds(r0, 4), :], src_st)
                pltpu.sync_copy(dst_hbm.at[pl.ds(r0, 4), :], dst_st)
                pltpu.sync_copy(vals_hbm.at[pl.ds(r0, 4), :], vals_st)

                @pl.loop(0, 4)
                def _(grp):
                    pltpu.async_copy(tbl.at[src_st.at[grp]], rows, gsem)
                    pltpu.make_async_copy(
                        tbl.at[src_st.at[grp]], rows, gsem).wait()

                    @pl.loop(0, 32)
                    def _(g):
                        v16 = vals_st[grp, pl.ds(g * 16, 16)]
                        base = g * 16
                        for i in range(16):
                            bc = lax.gather(
                                v16, bcast_idx[i], dnums, (1,),
                                mode=lax.GatherScatterMode.PROMISE_IN_BOUNDS)
                            rows[base + i, :] = rows[base + i, :] * bc

                    pltpu.async_copy(
                        rows, acc.at[dst_st.at[grp]], ssem, add=True)
                    pltpu.make_async_copy(
                        rows, acc.at[dst_st.at[grp]], ssem).wait()

            plsc.subcore_barrier()

            @pl.when(s < NUM_SUBCORES - 1)
            def _():
                pltpu.sync_copy(acc.at[pl.ds(lbase, rows_a)],
                                out_hbm.at[q, pl.ds(lbase, rows_a)])

            @pl.when(s == NUM_SUBCORES - 1)
            def _():
                pltpu.sync_copy(acc.at[pl.ds(lbase, rows_b)],
                                out_hbm.at[q, pl.ds(lbase, rows_b)])

            if p == 0:
                @pl.loop(0, 512)
                def _(i):
                    rows[i, :] = zv

                plsc.subcore_barrier()

    return sc_spmm


def _dense_layer(x, side4, w1, b1, w2, b2, n_nodes: int):
    bn = 1000
    grid = (n_nodes // bn,)

    def body(x_ref, s4_ref, w1_ref, b1_ref, w2_ref, b2_ref, y_ref, y4_ref):
        x_blk = x_ref[...]
        side = jnp.concatenate(
            [s4_ref[0], s4_ref[1], s4_ref[2], s4_ref[3]], axis=1)
        se = x_blk + side
        bi = x_blk * side
        h = lax.dot_general(se, w1_ref[...], (((1,), (1,)), ((), ())),
                            preferred_element_type=jnp.float32)
        h = h + lax.dot_general(bi, w2_ref[...], (((1,), (1,)), ((), ())),
                                preferred_element_type=jnp.float32)
        h = h + b1_ref[...] + b2_ref[...]
        y = jnp.where(h >= 0, h, jnp.float32(0.2) * h)
        n2 = jnp.sum(y * y, axis=1, keepdims=True)
        y = y * lax.rsqrt(jnp.maximum(n2, jnp.float32(1e-24)))
        y_ref[...] = y
        y4_ref[...] = jnp.stack(
            [y[:, 0:16], y[:, 16:32], y[:, 32:48], y[:, 48:64]], axis=0)

    return pl.pallas_call(
        body,
        grid=grid,
        in_specs=[
            pl.BlockSpec((bn, 64), lambda i: (i, 0)),
            pl.BlockSpec((4, bn, 16), lambda i: (0, i, 0)),
            pl.BlockSpec((64, 64), lambda i: (0, 0)),
            pl.BlockSpec((1, 64), lambda i: (0, 0)),
            pl.BlockSpec((64, 64), lambda i: (0, 0)),
            pl.BlockSpec((1, 64), lambda i: (0, 0)),
        ],
        out_specs=[
            pl.BlockSpec((bn, 64), lambda i: (i, 0)),
            pl.BlockSpec((4, bn, 16), lambda i: (0, i, 0)),
        ],
        out_shape=[
            jax.ShapeDtypeStruct((n_nodes, 64), jnp.float32),
            jax.ShapeDtypeStruct((4, n_nodes, 16), jnp.float32),
        ],
    )(x, side4, w1, b1.reshape(1, 64), w2, b2.reshape(1, 64))


def kernel(edge_vals, emb, W1, b1, W2, b2, edge_index):
    n_nodes, d = emb.shape
    n_edges = edge_vals.shape[0]
    n_layers = W1.shape[0]
    assert d == 64

    chunk = NUM_SUBCORES * 2048
    e_pad = ((n_edges + chunk - 1) // chunk) * chunk
    pad = e_pad - n_edges

    src = edge_index[0]
    dst = edge_index[1]
    vals = edge_vals
    if pad:
        src = jnp.concatenate([src, jnp.zeros((pad,), jnp.int32)])
        dst = jnp.concatenate([dst, jnp.full((pad,), n_nodes, jnp.int32)])
        vals = jnp.concatenate([vals, jnp.zeros((pad,), jnp.float32)])
    src = src.reshape(e_pad // 512, 512)
    dst = dst.reshape(e_pad // 512, 512)
    vals = vals.reshape(e_pad // 512, 512)

    sc_spmm = _make_sc_spmm(n_nodes, e_pad)

    x = emb
    x4 = jnp.stack(
        [emb[:, 0:16], emb[:, 16:32], emb[:, 32:48], emb[:, 48:64]])
    outs = [emb]
    for l in range(n_layers):
        side4 = sc_spmm(x4, src, dst, vals)
        x, x4 = _dense_layer(x, side4, W1[l], b1[l], W2[l], b2[l], n_nodes)
        outs.append(x)

    out = jnp.concatenate(outs, axis=1)
    half = n_nodes // 2
    return (out[:half], out[half:])


# speedup vs baseline: 1.3073x; 1.3073x over previous
import dataclasses
import functools

import jax
import jax.numpy as jnp
from jax import lax
from jax.experimental import pallas as pl
from jax.experimental.pallas import tpu as pltpu
from jax.experimental.pallas import tpu_sc as plsc

NUM_CORES = 2
NUM_SUBCORES = 16
LANES = 16


def _make_sc_spmm(n_nodes: int, e_pad: int):
    assert e_pad % (NUM_SUBCORES * 2048) == 0
    groups_per_tile = e_pad // 512 // NUM_SUBCORES
    stages = groups_per_tile // 4
    rows_a = ((n_nodes // NUM_SUBCORES) + 7) // 8 * 8
    rows_b = n_nodes - (NUM_SUBCORES - 1) * rows_a
    assert rows_b > 0 and rows_b % 8 == 0
    acc_rows = ((n_nodes + 1 + 16 * 448 - 1) // (16 * 448)) * (16 * 448)
    z_rows = acc_rows // NUM_SUBCORES // 448

    mesh = plsc.VectorSubcoreMesh(
        core_axis_name="c", subcore_axis_name="s",
        num_cores=NUM_CORES, num_subcores=NUM_SUBCORES)

    cp = pltpu.CompilerParams()
    if "needs_layout_passes" in pltpu.CompilerParams.__dataclass_fields__:
        cp = dataclasses.replace(cp, needs_layout_passes=False)
    if "use_tc_tiling_on_sc" in pltpu.CompilerParams.__dataclass_fields__:
        cp = dataclasses.replace(cp, use_tc_tiling_on_sc=False)

    @functools.partial(
        pl.kernel,
        compiler_params=cp,
        out_type=jax.ShapeDtypeStruct((4, n_nodes, 16), jnp.float32),
        mesh=mesh,
        scratch_types=[
            pltpu.VMEM((2, 4, 512), jnp.int32),
            pltpu.VMEM((2, 4, 512), jnp.int32),
            pltpu.VMEM((2, 4, 512), jnp.float32),
            pltpu.VMEM((512, 16), jnp.float32),
            pltpu.VMEM((512, 16), jnp.float32),
            pltpu.VMEM_SHARED((acc_rows, 16), jnp.float32),
            pltpu.VMEM_SHARED((acc_rows, 16), jnp.float32),
            pltpu.SemaphoreType.DMA,
            pltpu.SemaphoreType.DMA,
            pltpu.SemaphoreType.DMA,
            pltpu.SemaphoreType.DMA,
            pltpu.SemaphoreType.DMA,
        ],
    )
    def sc_spmm(x4_hbm, src_hbm, dst_hbm, vals_hbm, out_hbm,
                src_st, dst_st, vals_st, rows_a2, rows_b2, tbl, acc,
                gsa, gsb, ssa, ssb, isem):
        c = lax.axis_index("c")
        s = lax.axis_index("s")
        zv = jnp.zeros((LANES,), jnp.float32)
        lbase = s * rows_a

        dnums = lax.GatherDimensionNumbers(
            offset_dims=(), collapsed_slice_dims=(0,), start_index_map=(0,))
        bcast_idx = [jnp.full((LANES, 1), i, jnp.int32) for i in range(16)]

        def zero_rows(buf):
            @pl.loop(0, 512)
            def _(i):
                buf[i, :] = zv

        def scale(buf, par, grp):
            @pl.loop(0, 32)
            def _(g):
                v16 = vals_st[par, grp, pl.ds(g * 16, 16)]
                base = g * 16
                for i in range(16):
                    bc = lax.gather(
                        v16, bcast_idx[i], dnums, (1,),
                        mode=lax.GatherScatterMode.PROMISE_IN_BOUNDS)
                    buf[base + i, :] = buf[base + i, :] * bc

        def fire_g(buf, gs, par, grp):
            pltpu.async_copy(tbl.at[src_st.at[par, grp]], buf, gs)

        def wait_g(buf, gs):
            pltpu.make_async_copy(tbl.at[src_st.at[0, 0]], buf, gs).wait()

        def fire_s(buf, ss, par, grp):
            pltpu.async_copy(buf, acc.at[dst_st.at[par, grp]], ss, add=True)

        def wait_s(buf, ss):
            pltpu.make_async_copy(buf, acc.at[dst_st.at[0, 0]], ss).wait()

        row0 = s * groups_per_tile

        def fire_i(st, par):
            r0 = row0 + st * 4
            pltpu.async_copy(src_hbm.at[pl.ds(r0, 4), :], src_st.at[par], isem)
            pltpu.async_copy(dst_hbm.at[pl.ds(r0, 4), :], dst_st.at[par], isem)
            pltpu.async_copy(vals_hbm.at[pl.ds(r0, 4), :], vals_st.at[par],
                             isem)

        def wait_i():
            pltpu.make_async_copy(
                src_hbm.at[pl.ds(0, 4), :], src_st.at[0], isem).wait()
            pltpu.make_async_copy(
                dst_hbm.at[pl.ds(0, 4), :], dst_st.at[0], isem).wait()
            pltpu.make_async_copy(
                vals_hbm.at[pl.ds(0, 4), :], vals_st.at[0], isem).wait()

        zero_rows(rows_a2)

        for p in range(2):
            q = c * 2 + p

            @pl.when(s < NUM_SUBCORES - 1)
            def _():
                pltpu.sync_copy(x4_hbm.at[q, pl.ds(lbase, rows_a)],
                                tbl.at[pl.ds(lbase, rows_a)])

            @pl.when(s == NUM_SUBCORES - 1)
            def _():
                pltpu.sync_copy(x4_hbm.at[q, pl.ds(lbase, rows_b)],
                                tbl.at[pl.ds(lbase, rows_b)])

            zbase = s * (z_rows * 448)

            @pl.loop(0, z_rows)
            def _(k):
                pltpu.sync_copy(rows_a2.at[pl.ds(0, 448)],
                                acc.at[pl.ds(zbase + k * 448, 448)])

            plsc.subcore_barrier()

            fire_i(0, 0)
            wait_i()
            fire_g(rows_a2, gsa, 0, 0)

            @pl.loop(0, stages)
            def _(st):
                par = lax.bitwise_and(st, 1)
                nxt = lax.bitwise_and(st + 1, 1)

                wait_g(rows_a2, gsa)
                scale(rows_a2, par, 0)

                @pl.when(st > 0)
                def _():
                    wait_s(rows_b2, ssb)

                @pl.when(st < stages - 1)
                def _():
                    fire_i(st + 1, nxt)

                fire_s(rows_a2, ssa, par, 0)
                fire_g(rows_b2, gsb, par, 1)
                wait_g(rows_b2, gsb)
                scale(rows_b2, par, 1)
                wait_s(rows_a2, ssa)
                fire_s(rows_b2, ssb, par, 1)
                fire_g(rows_a2, gsa, par, 2)
                wait_g(rows_a2, gsa)
                scale(rows_a2, par, 2)
                wait_s(rows_b2, ssb)
                fire_s(rows_a2, ssa, par, 2)
                fire_g(rows_b2, gsb, par, 3)
                wait_g(rows_b2, gsb)
                scale(rows_b2, par, 3)
                wait_s(rows_a2, ssa)
                fire_s(rows_b2, ssb, par, 3)

                @pl.when(st < stages - 1)
                def _():
                    wait_i()
                    fire_g(rows_a2, gsa, nxt, 0)

            wait_s(rows_b2, ssb)
            plsc.subcore_barrier()

            @pl.when(s < NUM_SUBCORES - 1)
            def _():
                pltpu.sync_copy(acc.at[pl.ds(lbase, rows_a)],
                                out_hbm.at[q, pl.ds(lbase, rows_a)])

            @pl.when(s == NUM_SUBCORES - 1)
            def _():
                pltpu.sync_copy(acc.at[pl.ds(lbase, rows_b)],
                                out_hbm.at[q, pl.ds(lbase, rows_b)])

            if p == 0:
                zero_rows(rows_a2)
                plsc.subcore_barrier()

    return sc_spmm


def _dense_layer(x, side4, w1, b1, w2, b2, n_nodes: int):
    bn = 1000
    grid = (n_nodes // bn,)

    def body(x_ref, s4_ref, w1_ref, b1_ref, w2_ref, b2_ref, y_ref, y4_ref):
        x_blk = x_ref[...]
        side = jnp.concatenate(
            [s4_ref[0], s4_ref[1], s4_ref[2], s4_ref[3]], axis=1)
        se = x_blk + side
        bi = x_blk * side
        h = lax.dot_general(se, w1_ref[...], (((1,), (1,)), ((), ())),
                            preferred_element_type=jnp.float32)
        h = h + lax.dot_general(bi, w2_ref[...], (((1,), (1,)), ((), ())),
                                preferred_element_type=jnp.float32)
        h = h + b1_ref[...] + b2_ref[...]
        y = jnp.where(h >= 0, h, jnp.float32(0.2) * h)
        n2 = jnp.sum(y * y, axis=1, keepdims=True)
        y = y * lax.rsqrt(jnp.maximum(n2, jnp.float32(1e-24)))
        y_ref[...] = y
        y4_ref[...] = jnp.stack(
            [y[:, 0:16], y[:, 16:32], y[:, 32:48], y[:, 48:64]], axis=0)

    return pl.pallas_call(
        body,
        grid=grid,
        in_specs=[
            pl.BlockSpec((bn, 64), lambda i: (i, 0)),
            pl.BlockSpec((4, bn, 16), lambda i: (0, i, 0)),
            pl.BlockSpec((64, 64), lambda i: (0, 0)),
            pl.BlockSpec((1, 64), lambda i: (0, 0)),
            pl.BlockSpec((64, 64), lambda i: (0, 0)),
            pl.BlockSpec((1, 64), lambda i: (0, 0)),
        ],
        out_specs=[
            pl.BlockSpec((bn, 64), lambda i: (i, 0)),
            pl.BlockSpec((4, bn, 16), lambda i: (0, i, 0)),
        ],
        out_shape=[
            jax.ShapeDtypeStruct((n_nodes, 64), jnp.float32),
            jax.ShapeDtypeStruct((4, n_nodes, 16), jnp.float32),
        ],
    )(x, side4, w1, b1.reshape(1, 64), w2, b2.reshape(1, 64))


def kernel(edge_vals, emb, W1, b1, W2, b2, edge_index):
    n_nodes, d = emb.shape
    n_edges = edge_vals.shape[0]
    n_layers = W1.shape[0]
    assert d == 64

    chunk = NUM_SUBCORES * 2048
    e_pad = ((n_edges + chunk - 1) // chunk) * chunk
    pad = e_pad - n_edges

    src = edge_index[0]
    dst = edge_index[1]
    vals = edge_vals
    if pad:
        src = jnp.concatenate([src, jnp.zeros((pad,), jnp.int32)])
        dst = jnp.concatenate([dst, jnp.full((pad,), n_nodes, jnp.int32)])
        vals = jnp.concatenate([vals, jnp.zeros((pad,), jnp.float32)])
    src = src.reshape(e_pad // 512, 512)
    dst = dst.reshape(e_pad // 512, 512)
    vals = vals.reshape(e_pad // 512, 512)

    sc_spmm = _make_sc_spmm(n_nodes, e_pad)

    x = emb
    x4 = jnp.stack(
        [emb[:, 0:16], emb[:, 16:32], emb[:, 32:48], emb[:, 48:64]])
    outs = [emb]
    for l in range(n_layers):
        side4 = sc_spmm(x4, src, dst, vals)
        x, x4 = _dense_layer(x, side4, W1[l], b1[l], W2[l], b2[l], n_nodes)
        outs.append(x)

    out = jnp.concatenate(outs, axis=1)
    half = n_nodes // 2
    return (out[:half], out[half:])


# speedup vs baseline: 1.4205x; 1.0866x over previous
import dataclasses
import functools

import jax
import jax.numpy as jnp
from jax import lax
from jax.experimental import pallas as pl
from jax.experimental.pallas import tpu as pltpu
from jax.experimental.pallas import tpu_sc as plsc

NUM_CORES = 2
NUM_SUBCORES = 16
LANES = 16


def _make_sc_spmm(n_nodes: int, e_pad: int):
    assert e_pad % (NUM_SUBCORES * 2048) == 0
    groups_per_tile = e_pad // 128 // NUM_SUBCORES
    stages = groups_per_tile // 16
    rows_a = ((n_nodes // NUM_SUBCORES) + 7) // 8 * 8
    rows_b = n_nodes - (NUM_SUBCORES - 1) * rows_a
    assert rows_b > 0 and rows_b % 8 == 0
    acc_rows = ((n_nodes + 1 + 16 * 448 - 1) // (16 * 448)) * (16 * 448)
    zt_rows = acc_rows // NUM_SUBCORES
    z_full = zt_rows // 128
    z_rem = zt_rows - z_full * 128
    assert z_rem % 8 == 0

    mesh = plsc.VectorSubcoreMesh(
        core_axis_name="c", subcore_axis_name="s",
        num_cores=NUM_CORES, num_subcores=NUM_SUBCORES)

    cp = pltpu.CompilerParams()
    if "needs_layout_passes" in pltpu.CompilerParams.__dataclass_fields__:
        cp = dataclasses.replace(cp, needs_layout_passes=False)
    if "use_tc_tiling_on_sc" in pltpu.CompilerParams.__dataclass_fields__:
        cp = dataclasses.replace(cp, use_tc_tiling_on_sc=False)

    @functools.partial(
        pl.kernel,
        compiler_params=cp,
        out_type=jax.ShapeDtypeStruct((4, n_nodes, 16), jnp.float32),
        mesh=mesh,
        scratch_types=[
            pltpu.VMEM((2, 16, 128), jnp.int32),
            pltpu.VMEM((2, 16, 128), jnp.int32),
            pltpu.VMEM((2, 16, 128), jnp.float32),
            pltpu.VMEM((8, 128, 16), jnp.float32),
            pltpu.VMEM_SHARED((acc_rows, 16), jnp.float32),
            pltpu.VMEM_SHARED((acc_rows, 16), jnp.float32),
        ] + [pltpu.SemaphoreType.DMA] * 17,
    )
    def sc_spmm(x4_hbm, src_hbm, dst_hbm, vals_hbm, out_hbm,
                src_st, dst_st, vals_st, rows8, tbl, acc, *sems):
        gs = sems[0:8]
        ss = sems[8:16]
        isem = sems[16]
        c = lax.axis_index("c")
        s = lax.axis_index("s")
        zv = jnp.zeros((LANES,), jnp.float32)
        lbase = s * rows_a

        dnums = lax.GatherDimensionNumbers(
            offset_dims=(), collapsed_slice_dims=(0,), start_index_map=(0,))
        bcast_idx = [jnp.full((LANES, 1), i, jnp.int32) for i in range(16)]

        def zero_rows():
            @pl.loop(0, 128)
            def _(i):
                rows8[0, i, :] = zv

        def scale(b, par, slot):
            @pl.loop(0, 8)
            def _(g):
                v16 = vals_st[par, slot, pl.ds(g * 16, 16)]
                base = g * 16
                for i in range(16):
                    bc = lax.gather(
                        v16, bcast_idx[i], dnums, (1,),
                        mode=lax.GatherScatterMode.PROMISE_IN_BOUNDS)
                    rows8[b, base + i, :] = rows8[b, base + i, :] * bc

        def fire_g(b, par, slot):
            pltpu.async_copy(
                tbl.at[src_st.at[par, slot]], rows8.at[b], gs[b])

        def wait_g(b):
            pltpu.make_async_copy(
                tbl.at[src_st.at[0, 0]], rows8.at[b], gs[b]).wait()

        def fire_s(b, par, slot):
            pltpu.async_copy(
                rows8.at[b], acc.at[dst_st.at[par, slot]], ss[b], add=True)

        def wait_s(b):
            pltpu.make_async_copy(
                rows8.at[b], acc.at[dst_st.at[0, 0]], ss[b]).wait()

        row0 = s * groups_per_tile

        def fire_i(st, par):
            r0 = row0 + st * 16
            pltpu.async_copy(src_hbm.at[pl.ds(r0, 16), :], src_st.at[par],
                             isem)
            pltpu.async_copy(dst_hbm.at[pl.ds(r0, 16), :], dst_st.at[par],
                             isem)
            pltpu.async_copy(vals_hbm.at[pl.ds(r0, 16), :], vals_st.at[par],
                             isem)

        def wait_i():
            pltpu.make_async_copy(
                src_hbm.at[pl.ds(0, 16), :], src_st.at[0], isem).wait()
            pltpu.make_async_copy(
                dst_hbm.at[pl.ds(0, 16), :], dst_st.at[0], isem).wait()
            pltpu.make_async_copy(
                vals_hbm.at[pl.ds(0, 16), :], vals_st.at[0], isem).wait()

        zero_rows()

        for p in range(2):
            q = c * 2 + p

            @pl.when(s < NUM_SUBCORES - 1)
            def _():
                pltpu.sync_copy(x4_hbm.at[q, pl.ds(lbase, rows_a)],
                                tbl.at[pl.ds(lbase, rows_a)])

            @pl.when(s == NUM_SUBCORES - 1)
            def _():
                pltpu.sync_copy(x4_hbm.at[q, pl.ds(lbase, rows_b)],
                                tbl.at[pl.ds(lbase, rows_b)])

            zbase = s * zt_rows

            @pl.loop(0, z_full)
            def _(k):
                pltpu.sync_copy(rows8.at[0],
                                acc.at[pl.ds(zbase + k * 128, 128)])

            if z_rem:
                pltpu.sync_copy(
                    rows8.at[0, pl.ds(0, z_rem)],
                    acc.at[pl.ds(zbase + z_full * 128, z_rem)])

            plsc.subcore_barrier()

            fire_i(0, 0)
            wait_i()
            for k in range(4):
                fire_g(k, 0, k)

            @pl.loop(0, stages)
            def _(st):
                par = lax.bitwise_and(st, 1)
                nxt = lax.bitwise_and(st + 1, 1)

                for k in range(16):
                    b = k % 8
                    wait_g(b)
                    scale(b, par, k)
                    fire_s(b, par, k)
                    bn = (k + 4) % 8
                    if k < 4:
                        @pl.when(st > 0)
                        def _(bn=bn):
                            wait_s(bn)

                        @pl.when(st > 0)
                        def _(k=k, bn=bn):
                            fire_g(bn, par, k + 4)

                        @pl.when(st == 0)
                        def _(k=k, bn=bn):
                            fire_g(bn, par, k + 4)
                    elif k < 12:
                        wait_s(bn)
                        fire_g(bn, par, k + 4)
                    else:
                        wait_s(bn)

                        @pl.when(st < stages - 1)
                        def _(k=k, bn=bn, nxt=nxt):
                            fire_g(bn, nxt, k - 12)

                    if k == 4:
                        @pl.when(st < stages - 1)
                        def _(nxt=nxt):
                            fire_i(st + 1, nxt)

                    if k == 11:
                        @pl.when(st < stages - 1)
                        def _():
                            wait_i()

            for k in range(12, 16):
                wait_s(k % 8)
            plsc.subcore_barrier()

            @pl.when(s < NUM_SUBCORES - 1)
            def _():
                pltpu.sync_copy(acc.at[pl.ds(lbase, rows_a)],
                                out_hbm.at[q, pl.ds(lbase, rows_a)])

            @pl.when(s == NUM_SUBCORES - 1)
            def _():
                pltpu.sync_copy(acc.at[pl.ds(lbase, rows_b)],
                                out_hbm.at[q, pl.ds(lbase, rows_b)])

            if p == 0:
                zero_rows()
                plsc.subcore_barrier()

    return sc_spmm


def _dense_layer(x, side4, w1, b1, w2, b2, n_nodes: int):
    bn = 1000
    grid = (n_nodes // bn,)

    def body(x_ref, s4_ref, w1_ref, b1_ref, w2_ref, b2_ref, y_ref, y4_ref):
        x_blk = x_ref[...]
        side = jnp.concatenate(
            [s4_ref[0], s4_ref[1], s4_ref[2], s4_ref[3]], axis=1)
        se = x_blk + side
        bi = x_blk * side
        h = lax.dot_general(se, w1_ref[...], (((1,), (1,)), ((), ())),
                            preferred_element_type=jnp.float32)
        h = h + lax.dot_general(bi, w2_ref[...], (((1,), (1,)), ((), ())),
                                preferred_element_type=jnp.float32)
        h = h + b1_ref[...] + b2_ref[...]
        y = jnp.where(h >= 0, h, jnp.float32(0.2) * h)
        n2 = jnp.sum(y * y, axis=1, keepdims=True)
        y = y * lax.rsqrt(jnp.maximum(n2, jnp.float32(1e-24)))
        y_ref[...] = y
        y4_ref[...] = jnp.stack(
            [y[:, 0:16], y[:, 16:32], y[:, 32:48], y[:, 48:64]], axis=0)

    return pl.pallas_call(
        body,
        grid=grid,
        in_specs=[
            pl.BlockSpec((bn, 64), lambda i: (i, 0)),
            pl.BlockSpec((4, bn, 16), lambda i: (0, i, 0)),
            pl.BlockSpec((64, 64), lambda i: (0, 0)),
            pl.BlockSpec((1, 64), lambda i: (0, 0)),
            pl.BlockSpec((64, 64), lambda i: (0, 0)),
            pl.BlockSpec((1, 64), lambda i: (0, 0)),
        ],
        out_specs=[
            pl.BlockSpec((bn, 64), lambda i: (i, 0)),
            pl.BlockSpec((4, bn, 16), lambda i: (0, i, 0)),
        ],
        out_shape=[
            jax.ShapeDtypeStruct((n_nodes, 64), jnp.float32),
            jax.ShapeDtypeStruct((4, n_nodes, 16), jnp.float32),
        ],
    )(x, side4, w1, b1.reshape(1, 64), w2, b2.reshape(1, 64))


def kernel(edge_vals, emb, W1, b1, W2, b2, edge_index):
    n_nodes, d = emb.shape
    n_edges = edge_vals.shape[0]
    n_layers = W1.shape[0]
    assert d == 64

    chunk = NUM_SUBCORES * 2048
    e_pad = ((n_edges + chunk - 1) // chunk) * chunk
    pad = e_pad - n_edges

    src = edge_index[0]
    dst = edge_index[1]
    vals = edge_vals
    if pad:
        src = jnp.concatenate([src, jnp.zeros((pad,), jnp.int32)])
        dst = jnp.concatenate([dst, jnp.full((pad,), n_nodes, jnp.int32)])
        vals = jnp.concatenate([vals, jnp.zeros((pad,), jnp.float32)])
    src = src.reshape(e_pad // 128, 128)
    dst = dst.reshape(e_pad // 128, 128)
    vals = vals.reshape(e_pad // 128, 128)

    sc_spmm = _make_sc_spmm(n_nodes, e_pad)

    x = emb
    x4 = jnp.stack(
        [emb[:, 0:16], emb[:, 16:32], emb[:, 32:48], emb[:, 48:64]])
    outs = [emb]
    for l in range(n_layers):
        side4 = sc_spmm(x4, src, dst, vals)
        x, x4 = _dense_layer(x, side4, W1[l], b1[l], W2[l], b2[l], n_nodes)
        outs.append(x)

    out = jnp.concatenate(outs, axis=1)
    half = n_nodes // 2
    return (out[:half], out[half:])
